# baseline (speedup 1.0000x reference)
import jax
import jax.numpy as jnp
from jax import lax
from jax.experimental import pallas as pl
from jax.experimental.pallas import tpu as pltpu
from jax.experimental.pallas import tpu_sc as plsc

N = 10000
D = 128
E = 320000

NC = 2
NS = 16
NW = NC * NS
K = 128

E_PAD = ((E + NW * K - 1) // (NW * K)) * (NW * K)
EPT = E_PAD // NW
STEPS = EPT // K

N_ACC = 10016
RPT = 632
LAST = N - (NS - 1) * RPT

_mesh = plsc.VectorSubcoreMesh(core_axis_name="c", subcore_axis_name="s")


def _tile_rows_copy(src, dst, s):
  @pl.when(s < NS - 1)
  def _():
    pltpu.sync_copy(src.at[pl.ds(s * RPT, RPT)], dst.at[pl.ds(s * RPT, RPT)])

  @pl.when(s == NS - 1)
  def _():
    pltpu.sync_copy(src.at[pl.ds((NS - 1) * RPT, LAST)],
                    dst.at[pl.ds((NS - 1) * RPT, LAST)])


def _deg_body(row_hbm, ones_hbm, z_hbm, out0, out1,
              rowv0, rowv1, onesv, isem0, isem1, deg_sh):
  c = lax.axis_index("c")
  s = lax.axis_index("s")
  wid = s * NC + c
  base = wid * EPT
  _tile_rows_copy(z_hbm, deg_sh, s)
  pltpu.sync_copy(ones_hbm, onesv)
  plsc.subcore_barrier()

  pltpu.async_copy(row_hbm.at[pl.ds(base, K)], rowv0, isem0)

  def dbl(i, carry):
    t0 = 2 * i
    t1 = t0 + 1
    pltpu.async_copy(row_hbm.at[pl.ds(base + t1 * K, K)], rowv1, isem1)
    pltpu.make_async_copy(row_hbm.at[pl.ds(base, K)], rowv0, isem0).wait()
    pltpu.sync_copy(onesv, deg_sh.at[rowv0], add=True)

    @pl.when(t0 + 2 < STEPS)
    def _():
      pltpu.async_copy(row_hbm.at[pl.ds(base + (t0 + 2) * K, K)],
                       rowv0, isem0)

    pltpu.make_async_copy(row_hbm.at[pl.ds(base, K)], rowv1, isem1).wait()
    pltpu.sync_copy(onesv, deg_sh.at[rowv1], add=True)
    return carry

  lax.fori_loop(0, STEPS // 2, dbl, 0)
  if STEPS % 2:
    pltpu.make_async_copy(row_hbm.at[pl.ds(base, K)], rowv0, isem0).wait()
    pltpu.sync_copy(onesv, deg_sh.at[rowv0], add=True)
  plsc.subcore_barrier()

  @pl.when(c == 0)
  def _():
    _tile_rows_copy(deg_sh, out0, s)

  @pl.when(c == 1)
  def _():
    _tile_rows_copy(deg_sh, out1, s)


_deg_call = pl.kernel(
    _deg_body,
    out_type=(jax.ShapeDtypeStruct((N, D), jnp.float32),
              jax.ShapeDtypeStruct((N, D), jnp.float32)),
    mesh=_mesh,
    scratch_types=[
        pltpu.VMEM((K,), jnp.int32),
        pltpu.VMEM((K,), jnp.int32),
        pltpu.VMEM((K, D), jnp.float32),
        pltpu.SemaphoreType.DMA,
        pltpu.SemaphoreType.DMA,
        pltpu.VMEM_SHARED((N_ACC, D), jnp.float32),
    ],
)


NBUF = 3


def _edge_body(y_hbm, row_hbm, col_hbm, z_hbm, out0, out1,
               colv0, colv1, colv2,
               rowv0, rowv1, rowv2,
               rows0, rows1, rows2,
               csem0, csem1, csem2,
               isem0, isem1, isem2,
               gsem0, gsem1, gsem2,
               ssem0, ssem1, ssem2, agg_sh):
  colv = (colv0, colv1, colv2)
  rowv = (rowv0, rowv1, rowv2)
  rows = (rows0, rows1, rows2)
  csem = (csem0, csem1, csem2)
  isem = (isem0, isem1, isem2)
  gsem = (gsem0, gsem1, gsem2)
  ssem = (ssem0, ssem1, ssem2)
  c = lax.axis_index("c")
  s = lax.axis_index("s")
  wid = s * NC + c
  base = wid * EPT
  _tile_rows_copy(z_hbm, agg_sh, s)
  plsc.subcore_barrier()

  def load_idx(t, j):
    pltpu.async_copy(col_hbm.at[pl.ds(base + t * K, K)], colv[j], csem[j])
    pltpu.async_copy(row_hbm.at[pl.ds(base + t * K, K)], rowv[j], isem[j])

  def start_gather(j):
    pltpu.make_async_copy(col_hbm.at[pl.ds(base, K)], colv[j], csem[j]).wait()
    pltpu.async_copy(y_hbm.at[colv[j]], rows[j], gsem[j])

  def drain_scatter(j):
    pltpu.make_async_copy(rows[j], agg_sh.at[pl.ds(0, K)], ssem[j]).wait()

  load_idx(0, 0)
  load_idx(1, 1)
  start_gather(0)

  def trip(i, carry):
    for jj in range(NBUF):
      v = NBUF * i + jj

      @pl.when(v < STEPS)
      def _(jj=jj, v=v):
        pltpu.make_async_copy(y_hbm.at[colv[jj]], rows[jj], gsem[jj]).wait()
        pltpu.make_async_copy(row_hbm.at[pl.ds(base, K)],
                              rowv[jj], isem[jj]).wait()
        pltpu.async_copy(rows[jj], agg_sh.at[rowv[jj]], ssem[jj], add=True)
        ju = (jj + 2) % NBUF
        jn = (jj + 1) % NBUF

        @pl.when(v >= 1)
        def _drain(jj=jj):
          drain_scatter((jj + 2) % NBUF)

        @pl.when(v + 2 < STEPS)
        def _pref(v=v, ju=ju):
          load_idx(v + 2, ju)

        @pl.when(v + 1 < STEPS)
        def _gath(jn=jn):
          start_gather(jn)
    return carry

  lax.fori_loop(0, (STEPS + NBUF - 1) // NBUF, trip, 0)
  drain_scatter((STEPS - 1) % NBUF)

  plsc.subcore_barrier()

  @pl.when(c == 0)
  def _():
    _tile_rows_copy(agg_sh, out0, s)

  @pl.when(c == 1)
  def _():
    _tile_rows_copy(agg_sh, out1, s)


_edge_call = pl.kernel(
    _edge_body,
    out_type=(jax.ShapeDtypeStruct((N, D), jnp.float32),
              jax.ShapeDtypeStruct((N, D), jnp.float32)),
    mesh=_mesh,
    scratch_types=(
        [pltpu.VMEM((K,), jnp.int32) for _ in range(2 * NBUF)]
        + [pltpu.VMEM((K, D), jnp.float32) for _ in range(NBUF)]
        + [pltpu.SemaphoreType.DMA for _ in range(4 * NBUF)]
        + [pltpu.VMEM_SHARED((N_ACC, D), jnp.float32)]
    ),
)


GRID = 10
BR = N // GRID


def _dinv_body(d0_ref, d1_ref, o_ref):
  deg = 1.0 + d0_ref[:, 0:1] + d1_ref[:, 0:1]
  o_ref[...] = lax.rsqrt(deg)


def _dinv_call(d0, d1):
  return pl.pallas_call(
      _dinv_body,
      out_shape=jax.ShapeDtypeStruct((N, 1), jnp.float32),
  )(d0, d1)


def _pre_body(x_ref, w_ref, dinv_ref, xw_ref, y_ref):
  xw = jnp.dot(x_ref[...], w_ref[...], preferred_element_type=jnp.float32)
  xw_ref[...] = xw
  y_ref[...] = xw * dinv_ref[...]


def _pre_call(x, w, dinv):
  return pl.pallas_call(
      _pre_body,
      grid=(GRID,),
      in_specs=[
          pl.BlockSpec((BR, D), lambda i: (i, 0)),
          pl.BlockSpec((D, D), lambda i: (0, 0)),
          pl.BlockSpec((BR, 1), lambda i: (i, 0)),
      ],
      out_specs=[
          pl.BlockSpec((BR, D), lambda i: (i, 0)),
          pl.BlockSpec((BR, D), lambda i: (i, 0)),
      ],
      out_shape=[
          jax.ShapeDtypeStruct((N, D), jnp.float32),
          jax.ShapeDtypeStruct((N, D), jnp.float32),
      ],
  )(x, w, dinv)


def _conv_h(x_ref, xw_ref, a0_ref, a1_ref, dinv_ref, b_ref):
  dinv = dinv_ref[...]
  conv = dinv * (a0_ref[...] + a1_ref[...]) + (dinv * dinv) * xw_ref[...]
  return x_ref[...] + conv + b_ref[...]


def _post_mid_body(x_ref, xw_ref, a0_ref, a1_ref, dinv_ref, b_ref,
                   g_ref, be_ref, o_ref):
  h = _conv_h(x_ref, xw_ref, a0_ref, a1_ref, dinv_ref, b_ref)
  r = jnp.maximum(h, 0.0)
  mu = jnp.mean(r, axis=-1, keepdims=True)
  d = r - mu
  var = jnp.mean(d * d, axis=-1, keepdims=True)
  o_ref[...] = d * lax.rsqrt(var + 1e-5) * g_ref[...] + be_ref[...]


def _post_last_body(x_ref, xw_ref, a0_ref, a1_ref, dinv_ref, b_ref,
                    emb_ref, o_ref):
  h = _conv_h(x_ref, xw_ref, a0_ref, a1_ref, dinv_ref, b_ref)
  emb_ref[...] = h
  o_ref[...] = jnp.maximum(h, 0.0)


def _row_specs():
  return [
      pl.BlockSpec((BR, D), lambda i: (i, 0)),
      pl.BlockSpec((BR, D), lambda i: (i, 0)),
      pl.BlockSpec((BR, D), lambda i: (i, 0)),
      pl.BlockSpec((BR, D), lambda i: (i, 0)),
      pl.BlockSpec((BR, 1), lambda i: (i, 0)),
      pl.BlockSpec((1, D), lambda i: (0, 0)),
  ]


def _post_mid_call(x, xw, a0, a1, dinv, b, g, be):
  return pl.pallas_call(
      _post_mid_body,
      grid=(GRID,),
      in_specs=_row_specs() + [
          pl.BlockSpec((1, D), lambda i: (0, 0)),
          pl.BlockSpec((1, D), lambda i: (0, 0)),
      ],
      out_specs=pl.BlockSpec((BR, D), lambda i: (i, 0)),
      out_shape=jax.ShapeDtypeStruct((N, D), jnp.float32),
  )(x, xw, a0, a1, dinv, b, g, be)


def _post_last_call(x, xw, a0, a1, dinv, b):
  return pl.pallas_call(
      _post_last_body,
      grid=(GRID,),
      in_specs=_row_specs(),
      out_specs=[
          pl.BlockSpec((BR, D), lambda i: (i, 0)),
          pl.BlockSpec((BR, D), lambda i: (i, 0)),
      ],
      out_shape=[
          jax.ShapeDtypeStruct((N, D), jnp.float32),
          jax.ShapeDtypeStruct((N, D), jnp.float32),
      ],
  )(x, xw, a0, a1, dinv, b)


@jax.jit
def _run(x, edge_index, W0, b0, W1, b1, W2, b2, g0, be0, g1, be1):
  row = edge_index[0]
  col = edge_index[1]
  pad_iota = jnp.arange(E_PAD - E, dtype=row.dtype)
  rowp = jnp.concatenate([row, N + pad_iota % (N_ACC - N)])
  colp = jnp.concatenate([col, pad_iota % N])
  zeros_nd = jnp.zeros((N, D), jnp.float32)
  ones_k = jnp.ones((K, D), jnp.float32)

  d0, d1 = _deg_call(rowp, ones_k, zeros_nd)
  dinv = _dinv_call(d0, d1)

  Ws = (W0, W1, W2)
  bs = (b0, b1, b2)
  gs = (g0, g1)
  bes = (be0, be1)

  xc = x
  emb = xo = None
  for i in range(3):
    xw, y = _pre_call(xc, Ws[i], dinv)
    a0, a1 = _edge_call(y, rowp, colp, zeros_nd)
    b2d = bs[i].reshape(1, D)
    if i < 2:
      xc = _post_mid_call(xc, xw, a0, a1, dinv, b2d,
                          gs[i].reshape(1, D), bes[i].reshape(1, D))
    else:
      emb, xo = _post_last_call(xc, xw, a0, a1, dinv, b2d)
  return emb, xo


def kernel(x, edge_index, W0, b0, W1, b1, W2, b2, g0, be0, g1, be1):
  return _run(x, edge_index, W0, b0, W1, b1, W2, b2, g0, be0, g1, be1)


# speedup vs baseline: 1.1278x; 1.1278x over previous
import jax
import jax.numpy as jnp
from jax import lax
from jax.experimental import pallas as pl
from jax.experimental.pallas import tpu as pltpu
from jax.experimental.pallas import tpu_sc as plsc

N = 10000
D = 128
E = 320000

NC = 2
NS = 16
NW = NC * NS
K = 128

E_PAD = ((E + NW * K - 1) // (NW * K)) * (NW * K)
EPT = E_PAD // NW
STEPS = EPT // K

N_ACC = 10016
RPT = 632
LAST = N - (NS - 1) * RPT

_mesh = plsc.VectorSubcoreMesh(core_axis_name="c", subcore_axis_name="s")


def _tile_rows_copy(src, dst, s):
  @pl.when(s < NS - 1)
  def _():
    pltpu.sync_copy(src.at[pl.ds(s * RPT, RPT)], dst.at[pl.ds(s * RPT, RPT)])

  @pl.when(s == NS - 1)
  def _():
    pltpu.sync_copy(src.at[pl.ds((NS - 1) * RPT, LAST)],
                    dst.at[pl.ds((NS - 1) * RPT, LAST)])


def _deg_body(row_hbm, ones_hbm, z_hbm, out0, out1,
              rowv0, rowv1, onesv, isem0, isem1, deg_sh):
  c = lax.axis_index("c")
  s = lax.axis_index("s")
  wid = s * NC + c
  base = wid * EPT
  _tile_rows_copy(z_hbm, deg_sh, s)
  pltpu.sync_copy(ones_hbm, onesv)
  plsc.subcore_barrier()

  pltpu.async_copy(row_hbm.at[pl.ds(base, K)], rowv0, isem0)

  def dbl(i, carry):
    t0 = 2 * i
    t1 = t0 + 1
    pltpu.async_copy(row_hbm.at[pl.ds(base + t1 * K, K)], rowv1, isem1)
    pltpu.make_async_copy(row_hbm.at[pl.ds(base, K)], rowv0, isem0).wait()
    pltpu.sync_copy(onesv, deg_sh.at[rowv0], add=True)

    @pl.when(t0 + 2 < STEPS)
    def _():
      pltpu.async_copy(row_hbm.at[pl.ds(base + (t0 + 2) * K, K)],
                       rowv0, isem0)

    pltpu.make_async_copy(row_hbm.at[pl.ds(base, K)], rowv1, isem1).wait()
    pltpu.sync_copy(onesv, deg_sh.at[rowv1], add=True)
    return carry

  lax.fori_loop(0, STEPS // 2, dbl, 0)
  if STEPS % 2:
    pltpu.make_async_copy(row_hbm.at[pl.ds(base, K)], rowv0, isem0).wait()
    pltpu.sync_copy(onesv, deg_sh.at[rowv0], add=True)
  plsc.subcore_barrier()

  @pl.when(c == 0)
  def _():
    _tile_rows_copy(deg_sh, out0, s)

  @pl.when(c == 1)
  def _():
    _tile_rows_copy(deg_sh, out1, s)


_deg_call = pl.kernel(
    _deg_body,
    out_type=(jax.ShapeDtypeStruct((N, D), jnp.float32),
              jax.ShapeDtypeStruct((N, D), jnp.float32)),
    mesh=_mesh,
    scratch_types=[
        pltpu.VMEM((K,), jnp.int32),
        pltpu.VMEM((K,), jnp.int32),
        pltpu.VMEM((K, D), jnp.float32),
        pltpu.SemaphoreType.DMA,
        pltpu.SemaphoreType.DMA,
        pltpu.VMEM_SHARED((N_ACC, D), jnp.float32),
    ],
)


def _edge_body(y_hbm, row_hbm, col_hbm, z_hbm, out0, out1,
               colb, rowv0, rowv1, rows0, rows1,
               isem0, isem1, gsem0, gsem1, agg_sh):
  c = lax.axis_index("c")
  s = lax.axis_index("s")
  wid = s * NC + c
  base = wid * EPT
  _tile_rows_copy(z_hbm, agg_sh, s)
  pltpu.sync_copy(col_hbm.at[pl.ds(base, EPT)], colb)
  plsc.subcore_barrier()

  pltpu.async_copy(row_hbm.at[pl.ds(base, K)], rowv0, isem0)
  pltpu.async_copy(y_hbm.at[colb.at[pl.ds(0, K)]], rows0, gsem0)

  def dbl(i, carry):
    t0 = 2 * i
    t1 = t0 + 1
    pltpu.async_copy(row_hbm.at[pl.ds(base + t1 * K, K)], rowv1, isem1)
    pltpu.async_copy(y_hbm.at[colb.at[pl.ds(t1 * K, K)]], rows1, gsem1)
    pltpu.make_async_copy(y_hbm.at[colb.at[pl.ds(0, K)]], rows0, gsem0).wait()
    pltpu.make_async_copy(row_hbm.at[pl.ds(base, K)], rowv0, isem0).wait()
    pltpu.sync_copy(rows0, agg_sh.at[rowv0], add=True)

    @pl.when(t0 + 2 < STEPS)
    def _():
      pltpu.async_copy(row_hbm.at[pl.ds(base + (t0 + 2) * K, K)],
                       rowv0, isem0)
      pltpu.async_copy(y_hbm.at[colb.at[pl.ds((t0 + 2) * K, K)]],
                       rows0, gsem0)

    pltpu.make_async_copy(y_hbm.at[colb.at[pl.ds(0, K)]], rows1, gsem1).wait()
    pltpu.make_async_copy(row_hbm.at[pl.ds(base, K)], rowv1, isem1).wait()
    pltpu.sync_copy(rows1, agg_sh.at[rowv1], add=True)
    return carry

  lax.fori_loop(0, STEPS // 2, dbl, 0)
  if STEPS % 2:
    pltpu.make_async_copy(y_hbm.at[colb.at[pl.ds(0, K)]], rows0, gsem0).wait()
    pltpu.make_async_copy(row_hbm.at[pl.ds(base, K)], rowv0, isem0).wait()
    pltpu.sync_copy(rows0, agg_sh.at[rowv0], add=True)

  plsc.subcore_barrier()

  @pl.when(c == 0)
  def _():
    _tile_rows_copy(agg_sh, out0, s)

  @pl.when(c == 1)
  def _():
    _tile_rows_copy(agg_sh, out1, s)


_edge_call = pl.kernel(
    _edge_body,
    out_type=(jax.ShapeDtypeStruct((N, D), jnp.float32),
              jax.ShapeDtypeStruct((N, D), jnp.float32)),
    mesh=_mesh,
    scratch_types=[
        pltpu.VMEM((EPT,), jnp.int32),
        pltpu.VMEM((K,), jnp.int32),
        pltpu.VMEM((K,), jnp.int32),
        pltpu.VMEM((K, D), jnp.float32),
        pltpu.VMEM((K, D), jnp.float32),
        pltpu.SemaphoreType.DMA,
        pltpu.SemaphoreType.DMA,
        pltpu.SemaphoreType.DMA,
        pltpu.SemaphoreType.DMA,
        pltpu.VMEM_SHARED((N_ACC, D), jnp.float32),
    ],
)


GRID = 10
BR = N // GRID


def _dinv_body(d0_ref, d1_ref, o_ref):
  deg = 1.0 + d0_ref[:, 0:1] + d1_ref[:, 0:1]
  o_ref[...] = lax.rsqrt(deg)


def _dinv_call(d0, d1):
  return pl.pallas_call(
      _dinv_body,
      out_shape=jax.ShapeDtypeStruct((N, 1), jnp.float32),
  )(d0, d1)


def _pre_body(x_ref, w_ref, dinv_ref, xw_ref, y_ref):
  xw = jnp.dot(x_ref[...], w_ref[...], preferred_element_type=jnp.float32)
  xw_ref[...] = xw
  y_ref[...] = xw * dinv_ref[...]


def _pre_call(x, w, dinv):
  return pl.pallas_call(
      _pre_body,
      grid=(GRID,),
      in_specs=[
          pl.BlockSpec((BR, D), lambda i: (i, 0)),
          pl.BlockSpec((D, D), lambda i: (0, 0)),
          pl.BlockSpec((BR, 1), lambda i: (i, 0)),
      ],
      out_specs=[
          pl.BlockSpec((BR, D), lambda i: (i, 0)),
          pl.BlockSpec((BR, D), lambda i: (i, 0)),
      ],
      out_shape=[
          jax.ShapeDtypeStruct((N, D), jnp.float32),
          jax.ShapeDtypeStruct((N, D), jnp.float32),
      ],
  )(x, w, dinv)


def _conv_h(x_ref, xw_ref, a0_ref, a1_ref, dinv_ref, b_ref):
  dinv = dinv_ref[...]
  conv = dinv * (a0_ref[...] + a1_ref[...]) + (dinv * dinv) * xw_ref[...]
  return x_ref[...] + conv + b_ref[...]


def _post_mid_body(x_ref, xw_ref, a0_ref, a1_ref, dinv_ref, b_ref,
                   g_ref, be_ref, o_ref):
  h = _conv_h(x_ref, xw_ref, a0_ref, a1_ref, dinv_ref, b_ref)
  r = jnp.maximum(h, 0.0)
  mu = jnp.mean(r, axis=-1, keepdims=True)
  d = r - mu
  var = jnp.mean(d * d, axis=-1, keepdims=True)
  o_ref[...] = d * lax.rsqrt(var + 1e-5) * g_ref[...] + be_ref[...]


def _post_last_body(x_ref, xw_ref, a0_ref, a1_ref, dinv_ref, b_ref,
                    emb_ref, o_ref):
  h = _conv_h(x_ref, xw_ref, a0_ref, a1_ref, dinv_ref, b_ref)
  emb_ref[...] = h
  o_ref[...] = jnp.maximum(h, 0.0)


def _row_specs():
  return [
      pl.BlockSpec((BR, D), lambda i: (i, 0)),
      pl.BlockSpec((BR, D), lambda i: (i, 0)),
      pl.BlockSpec((BR, D), lambda i: (i, 0)),
      pl.BlockSpec((BR, D), lambda i: (i, 0)),
      pl.BlockSpec((BR, 1), lambda i: (i, 0)),
      pl.BlockSpec((1, D), lambda i: (0, 0)),
  ]


def _post_mid_call(x, xw, a0, a1, dinv, b, g, be):
  return pl.pallas_call(
      _post_mid_body,
      grid=(GRID,),
      in_specs=_row_specs() + [
          pl.BlockSpec((1, D), lambda i: (0, 0)),
          pl.BlockSpec((1, D), lambda i: (0, 0)),
      ],
      out_specs=pl.BlockSpec((BR, D), lambda i: (i, 0)),
      out_shape=jax.ShapeDtypeStruct((N, D), jnp.float32),
  )(x, xw, a0, a1, dinv, b, g, be)


def _post_last_call(x, xw, a0, a1, dinv, b):
  return pl.pallas_call(
      _post_last_body,
      grid=(GRID,),
      in_specs=_row_specs(),
      out_specs=[
          pl.BlockSpec((BR, D), lambda i: (i, 0)),
          pl.BlockSpec((BR, D), lambda i: (i, 0)),
      ],
      out_shape=[
          jax.ShapeDtypeStruct((N, D), jnp.float32),
          jax.ShapeDtypeStruct((N, D), jnp.float32),
      ],
  )(x, xw, a0, a1, dinv, b)


@jax.jit
def _run(x, edge_index, W0, b0, W1, b1, W2, b2, g0, be0, g1, be1):
  row = edge_index[0]
  col = edge_index[1]
  pad_iota = jnp.arange(E_PAD - E, dtype=row.dtype)
  rowp = jnp.concatenate([row, N + pad_iota % (N_ACC - N)])
  colp = jnp.concatenate([col, pad_iota % N])
  zeros_nd = jnp.zeros((N, D), jnp.float32)
  ones_k = jnp.ones((K, D), jnp.float32)

  d0, d1 = _deg_call(rowp, ones_k, zeros_nd)
  dinv = _dinv_call(d0, d1)

  Ws = (W0, W1, W2)
  bs = (b0, b1, b2)
  gs = (g0, g1)
  bes = (be0, be1)

  xc = x
  emb = xo = None
  for i in range(3):
    xw, y = _pre_call(xc, Ws[i], dinv)
    a0, a1 = _edge_call(y, rowp, colp, zeros_nd)
    b2d = bs[i].reshape(1, D)
    if i < 2:
      xc = _post_mid_call(xc, xw, a0, a1, dinv, b2d,
                          gs[i].reshape(1, D), bes[i].reshape(1, D))
    else:
      emb, xo = _post_last_call(xc, xw, a0, a1, dinv, b2d)
  return emb, xo


def kernel(x, edge_index, W0, b0, W1, b1, W2, b2, g0, be0, g1, be1):
  return _run(x, edge_index, W0, b0, W1, b1, W2, b2, g0, be0, g1, be1)


# speedup vs baseline: 1.1826x; 1.0487x over previous
import jax
import jax.numpy as jnp
from jax import lax
from jax.experimental import pallas as pl
from jax.experimental.pallas import tpu as pltpu
from jax.experimental.pallas import tpu_sc as plsc

N = 10000
D = 128
E = 320000

NC = 2
NS = 16
NW = NC * NS
K = 128

E_PAD = ((E + NW * K - 1) // (NW * K)) * (NW * K)
EPT = E_PAD // NW
STEPS = EPT // K

N_ACC = 10016
RPT = 632
LAST = N - (NS - 1) * RPT

_mesh = plsc.VectorSubcoreMesh(core_axis_name="c", subcore_axis_name="s")


def _tile_rows_copy(src, dst, s):
  @pl.when(s < NS - 1)
  def _():
    pltpu.sync_copy(src.at[pl.ds(s * RPT, RPT)], dst.at[pl.ds(s * RPT, RPT)])

  @pl.when(s == NS - 1)
  def _():
    pltpu.sync_copy(src.at[pl.ds((NS - 1) * RPT, LAST)],
                    dst.at[pl.ds((NS - 1) * RPT, LAST)])


def _deg_body(row_hbm, ones_hbm, z_hbm, out0, out1,
              rowv0, rowv1, onesv, isem0, isem1, deg_sh):
  c = lax.axis_index("c")
  s = lax.axis_index("s")
  wid = s * NC + c
  base = wid * EPT
  _tile_rows_copy(z_hbm, deg_sh, s)
  pltpu.sync_copy(ones_hbm, onesv)
  plsc.subcore_barrier()

  pltpu.async_copy(row_hbm.at[pl.ds(base, K)], rowv0, isem0)

  def dbl(i, carry):
    t0 = 2 * i
    t1 = t0 + 1
    pltpu.async_copy(row_hbm.at[pl.ds(base + t1 * K, K)], rowv1, isem1)
    pltpu.make_async_copy(row_hbm.at[pl.ds(base, K)], rowv0, isem0).wait()
    pltpu.sync_copy(onesv, deg_sh.at[rowv0], add=True)

    @pl.when(t0 + 2 < STEPS)
    def _():
      pltpu.async_copy(row_hbm.at[pl.ds(base + (t0 + 2) * K, K)],
                       rowv0, isem0)

    pltpu.make_async_copy(row_hbm.at[pl.ds(base, K)], rowv1, isem1).wait()
    pltpu.sync_copy(onesv, deg_sh.at[rowv1], add=True)
    return carry

  lax.fori_loop(0, STEPS // 2, dbl, 0)
  if STEPS % 2:
    pltpu.make_async_copy(row_hbm.at[pl.ds(base, K)], rowv0, isem0).wait()
    pltpu.sync_copy(onesv, deg_sh.at[rowv0], add=True)
  plsc.subcore_barrier()

  @pl.when(c == 0)
  def _():
    _tile_rows_copy(deg_sh, out0, s)

  @pl.when(c == 1)
  def _():
    _tile_rows_copy(deg_sh, out1, s)


_deg_call = pl.kernel(
    _deg_body,
    out_type=(jax.ShapeDtypeStruct((N, D), jnp.float32),
              jax.ShapeDtypeStruct((N, D), jnp.float32)),
    mesh=_mesh,
    scratch_types=[
        pltpu.VMEM((K,), jnp.int32),
        pltpu.VMEM((K,), jnp.int32),
        pltpu.VMEM((K, D), jnp.float32),
        pltpu.SemaphoreType.DMA,
        pltpu.SemaphoreType.DMA,
        pltpu.VMEM_SHARED((N_ACC, D), jnp.float32),
    ],
)


def _edge_body(y_hbm, row_hbm, col_hbm, z_hbm, out0, out1,
               colb, rowv0, rowv1, rows0, rows1,
               isem0, isem1, gsem0, gsem1, agg_sh):
  c = lax.axis_index("c")
  s = lax.axis_index("s")
  wid = s * NC + c
  base = wid * EPT
  _tile_rows_copy(z_hbm, agg_sh, s)
  pltpu.sync_copy(col_hbm.at[pl.ds(base, EPT)], colb)
  plsc.subcore_barrier()

  pltpu.async_copy(row_hbm.at[pl.ds(base, K)], rowv0, isem0)
  pltpu.async_copy(y_hbm.at[colb.at[pl.ds(0, K)]], rows0, gsem0)

  def dbl(i, carry):
    t0 = 2 * i
    t1 = t0 + 1
    pltpu.async_copy(row_hbm.at[pl.ds(base + t1 * K, K)], rowv1, isem1)
    pltpu.async_copy(y_hbm.at[colb.at[pl.ds(t1 * K, K)]], rows1, gsem1)
    pltpu.make_async_copy(y_hbm.at[colb.at[pl.ds(0, K)]], rows0, gsem0).wait()
    pltpu.make_async_copy(row_hbm.at[pl.ds(base, K)], rowv0, isem0).wait()
    pltpu.sync_copy(rows0, agg_sh.at[rowv0], add=True)

    @pl.when(t0 + 2 < STEPS)
    def _():
      pltpu.async_copy(row_hbm.at[pl.ds(base + (t0 + 2) * K, K)],
                       rowv0, isem0)
      pltpu.async_copy(y_hbm.at[colb.at[pl.ds((t0 + 2) * K, K)]],
                       rows0, gsem0)

    pltpu.make_async_copy(y_hbm.at[colb.at[pl.ds(0, K)]], rows1, gsem1).wait()
    pltpu.make_async_copy(row_hbm.at[pl.ds(base, K)], rowv1, isem1).wait()
    pltpu.sync_copy(rows1, agg_sh.at[rowv1], add=True)
    return carry

  lax.fori_loop(0, STEPS // 2, dbl, 0)
  if STEPS % 2:
    pltpu.make_async_copy(y_hbm.at[colb.at[pl.ds(0, K)]], rows0, gsem0).wait()
    pltpu.make_async_copy(row_hbm.at[pl.ds(base, K)], rowv0, isem0).wait()
    pltpu.sync_copy(rows0, agg_sh.at[rowv0], add=True)

  plsc.subcore_barrier()

  @pl.when(c == 0)
  def _():
    _tile_rows_copy(agg_sh, out0, s)

  @pl.when(c == 1)
  def _():
    _tile_rows_copy(agg_sh, out1, s)


_edge_call = pl.kernel(
    _edge_body,
    out_type=(jax.ShapeDtypeStruct((N, D), jnp.float32),
              jax.ShapeDtypeStruct((N, D), jnp.float32)),
    mesh=_mesh,
    scratch_types=[
        pltpu.VMEM((EPT,), jnp.int32),
        pltpu.VMEM((K,), jnp.int32),
        pltpu.VMEM((K,), jnp.int32),
        pltpu.VMEM((K, D), jnp.float32),
        pltpu.VMEM((K, D), jnp.float32),
        pltpu.SemaphoreType.DMA,
        pltpu.SemaphoreType.DMA,
        pltpu.SemaphoreType.DMA,
        pltpu.SemaphoreType.DMA,
        pltpu.VMEM_SHARED((N_ACC, D), jnp.float32),
    ],
)


GRID = 10
BR = N // GRID


def _pre0_body(x_ref, w_ref, d0_ref, d1_ref, xw_ref, y_ref, dinv_ref):
  deg = 1.0 + d0_ref[:, 0:1] + d1_ref[:, 0:1]
  dinv = lax.rsqrt(deg)
  dinv_ref[...] = dinv
  xw = jnp.dot(x_ref[...], w_ref[...], preferred_element_type=jnp.float32)
  xw_ref[...] = xw
  y_ref[...] = xw * dinv


def _pre0_call(x, w, d0, d1):
  return pl.pallas_call(
      _pre0_body,
      grid=(GRID,),
      in_specs=[
          pl.BlockSpec((BR, D), lambda i: (i, 0)),
          pl.BlockSpec((D, D), lambda i: (0, 0)),
          pl.BlockSpec((BR, D), lambda i: (i, 0)),
          pl.BlockSpec((BR, D), lambda i: (i, 0)),
      ],
      out_specs=[
          pl.BlockSpec((BR, D), lambda i: (i, 0)),
          pl.BlockSpec((BR, D), lambda i: (i, 0)),
          pl.BlockSpec((BR, 1), lambda i: (i, 0)),
      ],
      out_shape=[
          jax.ShapeDtypeStruct((N, D), jnp.float32),
          jax.ShapeDtypeStruct((N, D), jnp.float32),
          jax.ShapeDtypeStruct((N, 1), jnp.float32),
      ],
  )(x, w, d0, d1)


def _conv_h(x_ref, xw_ref, a0_ref, a1_ref, dinv_ref, b_ref):
  dinv = dinv_ref[...]
  conv = dinv * (a0_ref[...] + a1_ref[...]) + (dinv * dinv) * xw_ref[...]
  return x_ref[...] + conv + b_ref[...]


def _mid_body(x_ref, xw_ref, a0_ref, a1_ref, dinv_ref, b_ref,
              g_ref, be_ref, w_ref, xc_ref, xw2_ref, y2_ref):
  h = _conv_h(x_ref, xw_ref, a0_ref, a1_ref, dinv_ref, b_ref)
  r = jnp.maximum(h, 0.0)
  mu = jnp.mean(r, axis=-1, keepdims=True)
  d = r - mu
  var = jnp.mean(d * d, axis=-1, keepdims=True)
  xc = d * lax.rsqrt(var + 1e-5) * g_ref[...] + be_ref[...]
  xc_ref[...] = xc
  xw2 = jnp.dot(xc, w_ref[...], preferred_element_type=jnp.float32)
  xw2_ref[...] = xw2
  y2_ref[...] = xw2 * dinv_ref[...]


def _post_last_body(x_ref, xw_ref, a0_ref, a1_ref, dinv_ref, b_ref,
                    emb_ref, o_ref):
  h = _conv_h(x_ref, xw_ref, a0_ref, a1_ref, dinv_ref, b_ref)
  emb_ref[...] = h
  o_ref[...] = jnp.maximum(h, 0.0)


def _row_specs():
  return [
      pl.BlockSpec((BR, D), lambda i: (i, 0)),
      pl.BlockSpec((BR, D), lambda i: (i, 0)),
      pl.BlockSpec((BR, D), lambda i: (i, 0)),
      pl.BlockSpec((BR, D), lambda i: (i, 0)),
      pl.BlockSpec((BR, 1), lambda i: (i, 0)),
      pl.BlockSpec((1, D), lambda i: (0, 0)),
  ]


def _mid_call(x, xw, a0, a1, dinv, b, g, be, w_next):
  return pl.pallas_call(
      _mid_body,
      grid=(GRID,),
      in_specs=_row_specs() + [
          pl.BlockSpec((1, D), lambda i: (0, 0)),
          pl.BlockSpec((1, D), lambda i: (0, 0)),
          pl.BlockSpec((D, D), lambda i: (0, 0)),
      ],
      out_specs=[
          pl.BlockSpec((BR, D), lambda i: (i, 0)),
          pl.BlockSpec((BR, D), lambda i: (i, 0)),
          pl.BlockSpec((BR, D), lambda i: (i, 0)),
      ],
      out_shape=[
          jax.ShapeDtypeStruct((N, D), jnp.float32),
          jax.ShapeDtypeStruct((N, D), jnp.float32),
          jax.ShapeDtypeStruct((N, D), jnp.float32),
      ],
  )(x, xw, a0, a1, dinv, b, g, be, w_next)


def _post_last_call(x, xw, a0, a1, dinv, b):
  return pl.pallas_call(
      _post_last_body,
      grid=(GRID,),
      in_specs=_row_specs(),
      out_specs=[
          pl.BlockSpec((BR, D), lambda i: (i, 0)),
          pl.BlockSpec((BR, D), lambda i: (i, 0)),
      ],
      out_shape=[
          jax.ShapeDtypeStruct((N, D), jnp.float32),
          jax.ShapeDtypeStruct((N, D), jnp.float32),
      ],
  )(x, xw, a0, a1, dinv, b)


@jax.jit
def _run(x, edge_index, W0, b0, W1, b1, W2, b2, g0, be0, g1, be1):
  row = edge_index[0]
  col = edge_index[1]
  pad_iota = jnp.arange(E_PAD - E, dtype=row.dtype)
  rowp = jnp.concatenate([row, N + pad_iota % (N_ACC - N)])
  colp = jnp.concatenate([col, pad_iota % N])
  zeros_nd = jnp.zeros((N, D), jnp.float32)
  ones_k = jnp.ones((K, D), jnp.float32)

  d0, d1 = _deg_call(rowp, ones_k, zeros_nd)

  Ws = (W0, W1, W2)
  bs = (b0, b1, b2)
  gs = (g0, g1)
  bes = (be0, be1)

  xc = x
  xw, y, dinv = _pre0_call(xc, Ws[0], d0, d1)
  for i in range(3):
    a0, a1 = _edge_call(y, rowp, colp, zeros_nd)
    b2d = bs[i].reshape(1, D)
    if i < 2:
      xc, xw, y = _mid_call(xc, xw, a0, a1, dinv, b2d,
                            gs[i].reshape(1, D), bes[i].reshape(1, D),
                            Ws[i + 1])
    else:
      emb, xo = _post_last_call(xc, xw, a0, a1, dinv, b2d)
  return emb, xo


def kernel(x, edge_index, W0, b0, W1, b1, W2, b2, g0, be0, g1, be1):
  return _run(x, edge_index, W0, b0, W1, b1, W2, b2, g0, be0, g1, be1)


# speedup vs baseline: 1.1978x; 1.0128x over previous
import jax
import jax.numpy as jnp
from jax import lax
from jax.experimental import pallas as pl
from jax.experimental.pallas import tpu as pltpu
from jax.experimental.pallas import tpu_sc as plsc

N = 10000
D = 128
E = 320000

NC = 2
NS = 16
NW = NC * NS
K = 128

E_PAD = ((E + NW * K - 1) // (NW * K)) * (NW * K)
EPT = E_PAD // NW
STEPS = EPT // K

N_ACC = 10016
RPT = 632
LAST = N - (NS - 1) * RPT

_mesh = plsc.VectorSubcoreMesh(core_axis_name="c", subcore_axis_name="s")


def _tile_rows_copy(src, dst, s):
  @pl.when(s < NS - 1)
  def _():
    pltpu.sync_copy(src.at[pl.ds(s * RPT, RPT)], dst.at[pl.ds(s * RPT, RPT)])

  @pl.when(s == NS - 1)
  def _():
    pltpu.sync_copy(src.at[pl.ds((NS - 1) * RPT, LAST)],
                    dst.at[pl.ds((NS - 1) * RPT, LAST)])


def _deg_body(row_hbm, ones_hbm, z_hbm, out0, out1,
              rowv0, rowv1, onesv, isem0, isem1, deg_sh):
  c = lax.axis_index("c")
  s = lax.axis_index("s")
  wid = s * NC + c
  base = wid * EPT
  _tile_rows_copy(z_hbm, deg_sh, s)
  pltpu.sync_copy(ones_hbm, onesv)
  plsc.subcore_barrier()

  pltpu.async_copy(row_hbm.at[pl.ds(base, K)], rowv0, isem0)

  def dbl(i, carry):
    t0 = 2 * i
    t1 = t0 + 1
    pltpu.async_copy(row_hbm.at[pl.ds(base + t1 * K, K)], rowv1, isem1)
    pltpu.make_async_copy(row_hbm.at[pl.ds(base, K)], rowv0, isem0).wait()
    pltpu.sync_copy(onesv, deg_sh.at[rowv0], add=True)

    @pl.when(t0 + 2 < STEPS)
    def _():
      pltpu.async_copy(row_hbm.at[pl.ds(base + (t0 + 2) * K, K)],
                       rowv0, isem0)

    pltpu.make_async_copy(row_hbm.at[pl.ds(base, K)], rowv1, isem1).wait()
    pltpu.sync_copy(onesv, deg_sh.at[rowv1], add=True)
    return carry

  lax.fori_loop(0, STEPS // 2, dbl, 0)
  if STEPS % 2:
    pltpu.make_async_copy(row_hbm.at[pl.ds(base, K)], rowv0, isem0).wait()
    pltpu.sync_copy(onesv, deg_sh.at[rowv0], add=True)
  plsc.subcore_barrier()

  @pl.when(c == 0)
  def _():
    _tile_rows_copy(deg_sh, out0, s)

  @pl.when(c == 1)
  def _():
    _tile_rows_copy(deg_sh, out1, s)


_deg_call = pl.kernel(
    _deg_body,
    out_type=(jax.ShapeDtypeStruct((N, D), jnp.float32),
              jax.ShapeDtypeStruct((N, D), jnp.float32)),
    mesh=_mesh,
    scratch_types=[
        pltpu.VMEM((K,), jnp.int32),
        pltpu.VMEM((K,), jnp.int32),
        pltpu.VMEM((K, D), jnp.float32),
        pltpu.SemaphoreType.DMA,
        pltpu.SemaphoreType.DMA,
        pltpu.VMEM_SHARED((N_ACC, D), jnp.float32),
    ],
)


def _edge_body(y_hbm, row_hbm, col_hbm, z_hbm, out0, out1,
               colb, rowv0, rowv1, rows0, rows1,
               isem0, isem1, gsem0, gsem1, agg_sh):
  c = lax.axis_index("c")
  s = lax.axis_index("s")
  wid = s * NC + c
  base = wid * EPT
  _tile_rows_copy(z_hbm, agg_sh, s)
  pltpu.sync_copy(col_hbm.at[pl.ds(base, EPT)], colb)
  plsc.subcore_barrier()

  pltpu.async_copy(row_hbm.at[pl.ds(base, K)], rowv0, isem0)
  pltpu.async_copy(y_hbm.at[colb.at[pl.ds(0, K)]], rows0, gsem0)

  def dbl(i, carry):
    t0 = 2 * i
    t1 = t0 + 1
    pltpu.async_copy(row_hbm.at[pl.ds(base + t1 * K, K)], rowv1, isem1)
    pltpu.async_copy(y_hbm.at[colb.at[pl.ds(t1 * K, K)]], rows1, gsem1)
    pltpu.make_async_copy(y_hbm.at[colb.at[pl.ds(0, K)]], rows0, gsem0).wait()
    pltpu.make_async_copy(row_hbm.at[pl.ds(base, K)], rowv0, isem0).wait()
    pltpu.sync_copy(rows0, agg_sh.at[rowv0], add=True)

    @pl.when(t0 + 2 < STEPS)
    def _():
      pltpu.async_copy(row_hbm.at[pl.ds(base + (t0 + 2) * K, K)],
                       rowv0, isem0)
      pltpu.async_copy(y_hbm.at[colb.at[pl.ds((t0 + 2) * K, K)]],
                       rows0, gsem0)

    pltpu.make_async_copy(y_hbm.at[colb.at[pl.ds(0, K)]], rows1, gsem1).wait()
    pltpu.make_async_copy(row_hbm.at[pl.ds(base, K)], rowv1, isem1).wait()
    pltpu.sync_copy(rows1, agg_sh.at[rowv1], add=True)
    return carry

  lax.fori_loop(0, STEPS // 2, dbl, 0)
  if STEPS % 2:
    pltpu.make_async_copy(y_hbm.at[colb.at[pl.ds(0, K)]], rows0, gsem0).wait()
    pltpu.make_async_copy(row_hbm.at[pl.ds(base, K)], rowv0, isem0).wait()
    pltpu.sync_copy(rows0, agg_sh.at[rowv0], add=True)

  plsc.subcore_barrier()

  @pl.when(c == 0)
  def _():
    _tile_rows_copy(agg_sh, out0, s)

  @pl.when(c == 1)
  def _():
    _tile_rows_copy(agg_sh, out1, s)


_edge_call = pl.kernel(
    _edge_body,
    out_type=(jax.ShapeDtypeStruct((N, D), jnp.float32),
              jax.ShapeDtypeStruct((N, D), jnp.float32)),
    mesh=_mesh,
    scratch_types=[
        pltpu.VMEM((EPT,), jnp.int32),
        pltpu.VMEM((K,), jnp.int32),
        pltpu.VMEM((K,), jnp.int32),
        pltpu.VMEM((K, D), jnp.float32),
        pltpu.VMEM((K, D), jnp.float32),
        pltpu.SemaphoreType.DMA,
        pltpu.SemaphoreType.DMA,
        pltpu.SemaphoreType.DMA,
        pltpu.SemaphoreType.DMA,
        pltpu.VMEM_SHARED((N_ACC, D), jnp.float32),
    ],
)


GRID = 10
BR = N // GRID


def _pre0_body(x_ref, w_ref, d0_ref, d1_ref, b_ref, u_ref, y_ref, dinv_ref):
  deg = 1.0 + d0_ref[:, 0:1] + d1_ref[:, 0:1]
  dinv = lax.rsqrt(deg)
  dinv_ref[...] = dinv
  x = x_ref[...]
  xw = jnp.dot(x, w_ref[...], preferred_element_type=jnp.float32)
  u_ref[...] = x + (dinv * dinv) * xw + b_ref[...]
  y_ref[...] = xw * dinv


def _pre0_call(x, w, d0, d1, b):
  return pl.pallas_call(
      _pre0_body,
      grid=(GRID,),
      in_specs=[
          pl.BlockSpec((BR, D), lambda i: (i, 0)),
          pl.BlockSpec((D, D), lambda i: (0, 0)),
          pl.BlockSpec((BR, D), lambda i: (i, 0)),
          pl.BlockSpec((BR, D), lambda i: (i, 0)),
          pl.BlockSpec((1, D), lambda i: (0, 0)),
      ],
      out_specs=[
          pl.BlockSpec((BR, D), lambda i: (i, 0)),
          pl.BlockSpec((BR, D), lambda i: (i, 0)),
          pl.BlockSpec((BR, 1), lambda i: (i, 0)),
      ],
      out_shape=[
          jax.ShapeDtypeStruct((N, D), jnp.float32),
          jax.ShapeDtypeStruct((N, D), jnp.float32),
          jax.ShapeDtypeStruct((N, 1), jnp.float32),
      ],
  )(x, w, d0, d1, b)


def _mid_body(u_ref, a0_ref, a1_ref, dinv_ref, g_ref, be_ref, b2_ref,
              w_ref, u2_ref, y2_ref):
  dinv = dinv_ref[...]
  h = u_ref[...] + dinv * (a0_ref[...] + a1_ref[...])
  r = jnp.maximum(h, 0.0)
  mu = jnp.mean(r, axis=-1, keepdims=True)
  d = r - mu
  var = jnp.mean(d * d, axis=-1, keepdims=True)
  xc = d * lax.rsqrt(var + 1e-5) * g_ref[...] + be_ref[...]
  xw2 = jnp.dot(xc, w_ref[...], preferred_element_type=jnp.float32)
  u2_ref[...] = xc + (dinv * dinv) * xw2 + b2_ref[...]
  y2_ref[...] = xw2 * dinv


def _post_last_body(u_ref, a0_ref, a1_ref, dinv_ref, emb_ref, o_ref):
  h = u_ref[...] + dinv_ref[...] * (a0_ref[...] + a1_ref[...])
  emb_ref[...] = h
  o_ref[...] = jnp.maximum(h, 0.0)


def _row_specs():
  return [
      pl.BlockSpec((BR, D), lambda i: (i, 0)),
      pl.BlockSpec((BR, D), lambda i: (i, 0)),
      pl.BlockSpec((BR, D), lambda i: (i, 0)),
      pl.BlockSpec((BR, 1), lambda i: (i, 0)),
  ]


def _mid_call(u, a0, a1, dinv, g, be, b_next, w_next):
  return pl.pallas_call(
      _mid_body,
      grid=(GRID,),
      in_specs=_row_specs() + [
          pl.BlockSpec((1, D), lambda i: (0, 0)),
          pl.BlockSpec((1, D), lambda i: (0, 0)),
          pl.BlockSpec((1, D), lambda i: (0, 0)),
          pl.BlockSpec((D, D), lambda i: (0, 0)),
      ],
      out_specs=[
          pl.BlockSpec((BR, D), lambda i: (i, 0)),
          pl.BlockSpec((BR, D), lambda i: (i, 0)),
      ],
      out_shape=[
          jax.ShapeDtypeStruct((N, D), jnp.float32),
          jax.ShapeDtypeStruct((N, D), jnp.float32),
      ],
  )(u, a0, a1, dinv, g, be, b_next, w_next)


def _post_last_call(u, a0, a1, dinv):
  return pl.pallas_call(
      _post_last_body,
      grid=(GRID,),
      in_specs=_row_specs(),
      out_specs=[
          pl.BlockSpec((BR, D), lambda i: (i, 0)),
          pl.BlockSpec((BR, D), lambda i: (i, 0)),
      ],
      out_shape=[
          jax.ShapeDtypeStruct((N, D), jnp.float32),
          jax.ShapeDtypeStruct((N, D), jnp.float32),
      ],
  )(u, a0, a1, dinv)


@jax.jit
def _run(x, edge_index, W0, b0, W1, b1, W2, b2, g0, be0, g1, be1):
  row = edge_index[0]
  col = edge_index[1]
  pad_iota = jnp.arange(E_PAD - E, dtype=row.dtype)
  rowp = jnp.concatenate([row, N + pad_iota % (N_ACC - N)])
  colp = jnp.concatenate([col, pad_iota % N])
  zeros_nd = jnp.zeros((N, D), jnp.float32)
  ones_k = jnp.ones((K, D), jnp.float32)

  d0, d1 = _deg_call(rowp, ones_k, zeros_nd)

  Ws = (W0, W1, W2)
  bs = (b0, b1, b2)
  gs = (g0, g1)
  bes = (be0, be1)

  u, y, dinv = _pre0_call(x, Ws[0], d0, d1, bs[0].reshape(1, D))
  for i in range(3):
    a0, a1 = _edge_call(y, rowp, colp, zeros_nd)
    if i < 2:
      u, y = _mid_call(u, a0, a1, dinv,
                       gs[i].reshape(1, D), bes[i].reshape(1, D),
                       bs[i + 1].reshape(1, D), Ws[i + 1])
    else:
      emb, xo = _post_last_call(u, a0, a1, dinv)
  return emb, xo


def kernel(x, edge_index, W0, b0, W1, b1, W2, b2, g0, be0, g1, be1):
  return _run(x, edge_index, W0, b0, W1, b1, W2, b2, g0, be0, g1, be1)
